# Initial kernel scaffold; baseline (speedup 1.0000x reference)
import jax
import jax.numpy as jnp
from jax.experimental import pallas as pl


def kernel(features, cluster_centers):
    raise NotImplementedError("write your pallas kernel here")


# speedup vs baseline: 1.5265x; 1.5265x over previous
import jax
import jax.numpy as jnp
from jax.experimental import pallas as pl

Q = 16384
K = 8192
D = 256
BQ = 256
WBOUNDS = (0, 2048, 4096, 6144, 8192)


def _sumsq_body(a_ref, o_ref):
    a = a_ref[...]
    o_ref[...] = jnp.sum(a * a, axis=1, keepdims=True)


def _lexmin(m1, i1, m2, i2):
    take1 = (m1 < m2) | ((m1 == m2) & (i1 < i2))
    return jnp.where(take1, m1, m2), jnp.where(take1, i1, i2)


def _argmin_body(xb_ref, cb2_ref, x2_ref, c2_ref, o_ref):
    xb = xb_ref[...]
    cb2 = cb2_ref[...]
    mm2 = jax.lax.dot_general(
        xb, cb2, (((1,), (1,)), ((), ())),
        preferred_element_type=jnp.float32)
    d2 = (x2_ref[...] + c2_ref[...]) - mm2

    wmins = []
    wargs = []
    for w in range(len(WBOUNDS) - 1):
        lo, hi = WBOUNDS[w], WBOUNDS[w + 1]
        alo = -(-lo // 128) * 128
        ahi = (hi // 128) * 128
        mi = jnp.min(d2[:, alo:ahi], axis=1)
        ii = jnp.argmin(d2[:, alo:ahi], axis=1).astype(jnp.int32) + alo
        if lo % 128:
            pb = (lo // 128) * 128
            cols = pb + jax.lax.broadcasted_iota(jnp.int32, (BQ, 128), 1)
            pv = jnp.where(cols >= lo, d2[:, pb:pb + 128], jnp.inf)
            mp = jnp.min(pv, axis=1)
            ip = jnp.argmin(pv, axis=1).astype(jnp.int32) + pb
            mi, ii = _lexmin(mp, ip, mi, ii)
        if hi % 128:
            pb = ahi
            cols = pb + jax.lax.broadcasted_iota(jnp.int32, (BQ, 128), 1)
            pv = jnp.where(cols < hi, d2[:, pb:pb + 128], jnp.inf)
            mp = jnp.min(pv, axis=1)
            ip = jnp.argmin(pv, axis=1).astype(jnp.int32) + pb
            mi, ii = _lexmin(mi, ii, mp, ip)
        wmins.append(jnp.sqrt(jnp.maximum(mi, 0.0)))
        wargs.append(ii)

    bacc = wmins[0].astype(jnp.bfloat16).astype(jnp.float32)
    iacc = wargs[0]
    for w in range(1, len(WBOUNDS) - 1):
        take = wmins[w] < bacc
        iacc = jnp.where(take, wargs[w], iacc)
        nxt = jnp.where(take, wmins[w], bacc)
        bacc = nxt.astype(jnp.bfloat16).astype(jnp.float32)

    o_ref[...] = iacc.reshape(1, 1, BQ)


def kernel(features, cluster_centers):
    x2 = pl.pallas_call(
        _sumsq_body,
        out_shape=jax.ShapeDtypeStruct((Q, 1), jnp.float32),
    )(features)
    c2col = pl.pallas_call(
        _sumsq_body,
        out_shape=jax.ShapeDtypeStruct((K, 1), jnp.float32),
    )(cluster_centers)
    c2row = c2col.reshape(K)[None, :]

    xb = features.astype(jnp.bfloat16)
    cb2 = cluster_centers.astype(jnp.bfloat16) * 2

    out = pl.pallas_call(
        _argmin_body,
        grid=(Q // BQ,),
        in_specs=[
            pl.BlockSpec((BQ, D), lambda i: (i, 0)),
            pl.BlockSpec((K, D), lambda i: (0, 0)),
            pl.BlockSpec((BQ, 1), lambda i: (i, 0)),
            pl.BlockSpec((1, K), lambda i: (0, 0)),
        ],
        out_specs=pl.BlockSpec((1, 1, BQ), lambda i: (i, 0, 0)),
        out_shape=jax.ShapeDtypeStruct((Q // BQ, 1, BQ), jnp.int32),
    )(xb, cb2, x2, c2row)
    return out.reshape(Q)


# speedup vs baseline: 1.7236x; 1.1292x over previous
import jax
import jax.numpy as jnp
from jax.experimental import pallas as pl

Q = 16384
K = 8192
D = 256
BQ = 256
WBOUNDS = (0, 2048, 4096, 6144, 8192)


def _sumsq_body(a_ref, o_ref):
    a = a_ref[...]
    o_ref[...] = jnp.sum(a * a, axis=1, keepdims=True)


def _argmin_body(xb_ref, cb2_ref, x2_ref, c2_ref, o_ref):
    xb = xb_ref[...]
    cb2 = cb2_ref[...]
    mm2 = jax.lax.dot_general(
        xb, cb2, (((1,), (1,)), ((), ())),
        preferred_element_type=jnp.float32)
    score = c2_ref[...] - mm2

    x2 = x2_ref[...].reshape(BQ)
    wd = []
    wi = []
    for w in range(len(WBOUNDS) - 1):
        lo, hi = WBOUNDS[w], WBOUNDS[w + 1]
        seg = score[:, lo:hi]
        m = jnp.min(seg, axis=1)
        i = jnp.argmin(seg, axis=1).astype(jnp.int32) + lo
        d2 = jnp.maximum(x2 + m, 0.0)
        wd.append(jnp.sqrt(d2))
        wi.append(i)

    bacc = wd[0].astype(jnp.bfloat16).astype(jnp.float32)
    iacc = wi[0]
    for w in range(1, len(WBOUNDS) - 1):
        take = wd[w] < bacc
        iacc = jnp.where(take, wi[w], iacc)
        nxt = jnp.where(take, wd[w], bacc)
        bacc = nxt.astype(jnp.bfloat16).astype(jnp.float32)

    o_ref[...] = iacc.reshape(1, 1, BQ)


def kernel(features, cluster_centers):
    x2 = pl.pallas_call(
        _sumsq_body,
        out_shape=jax.ShapeDtypeStruct((Q, 1), jnp.float32),
    )(features)
    c2col = pl.pallas_call(
        _sumsq_body,
        out_shape=jax.ShapeDtypeStruct((K, 1), jnp.float32),
    )(cluster_centers)
    c2row = c2col.reshape(K)[None, :]

    xb = features.astype(jnp.bfloat16)
    cb2 = cluster_centers.astype(jnp.bfloat16) * 2

    out = pl.pallas_call(
        _argmin_body,
        grid=(Q // BQ,),
        in_specs=[
            pl.BlockSpec((BQ, D), lambda i: (i, 0)),
            pl.BlockSpec((K, D), lambda i: (0, 0)),
            pl.BlockSpec((BQ, 1), lambda i: (i, 0)),
            pl.BlockSpec((1, K), lambda i: (0, 0)),
        ],
        out_specs=pl.BlockSpec((1, 1, BQ), lambda i: (i, 0, 0)),
        out_shape=jax.ShapeDtypeStruct((Q // BQ, 1, BQ), jnp.int32),
    )(xb, cb2, x2, c2row)
    return out.reshape(Q)


# speedup vs baseline: 1.7244x; 1.0004x over previous
import jax
import jax.numpy as jnp
from jax.experimental import pallas as pl

Q = 16384
K = 8192
D = 256
BQ = 512
WBOUNDS = (0, 2048, 4096, 6144, 8192)


def _sumsq_body(a_ref, o_ref):
    a = a_ref[...]
    o_ref[...] = jnp.sum(a * a, axis=1, keepdims=True)


def _argmin_body(xb_ref, cb2_ref, x2_ref, c2_ref, o_ref):
    xb = xb_ref[...]
    cb2 = cb2_ref[...]
    mm2 = jax.lax.dot_general(
        xb, cb2, (((1,), (1,)), ((), ())),
        preferred_element_type=jnp.float32)
    score = c2_ref[...] - mm2

    x2 = x2_ref[...].reshape(BQ)
    wd = []
    wi = []
    for w in range(len(WBOUNDS) - 1):
        lo, hi = WBOUNDS[w], WBOUNDS[w + 1]
        seg = score[:, lo:hi]
        m = jnp.min(seg, axis=1)
        i = jnp.argmin(seg, axis=1).astype(jnp.int32) + lo
        d2 = jnp.maximum(x2 + m, 0.0)
        wd.append(jnp.sqrt(d2))
        wi.append(i)

    bacc = wd[0].astype(jnp.bfloat16).astype(jnp.float32)
    iacc = wi[0]
    for w in range(1, len(WBOUNDS) - 1):
        take = wd[w] < bacc
        iacc = jnp.where(take, wi[w], iacc)
        nxt = jnp.where(take, wd[w], bacc)
        bacc = nxt.astype(jnp.bfloat16).astype(jnp.float32)

    o_ref[...] = iacc.reshape(1, 1, BQ)


def kernel(features, cluster_centers):
    x2 = pl.pallas_call(
        _sumsq_body,
        out_shape=jax.ShapeDtypeStruct((Q, 1), jnp.float32),
    )(features)
    c2col = pl.pallas_call(
        _sumsq_body,
        out_shape=jax.ShapeDtypeStruct((K, 1), jnp.float32),
    )(cluster_centers)
    c2row = c2col.reshape(K)[None, :]

    xb = features.astype(jnp.bfloat16)
    cb2 = cluster_centers.astype(jnp.bfloat16) * 2

    out = pl.pallas_call(
        _argmin_body,
        grid=(Q // BQ,),
        in_specs=[
            pl.BlockSpec((BQ, D), lambda i: (i, 0)),
            pl.BlockSpec((K, D), lambda i: (0, 0)),
            pl.BlockSpec((BQ, 1), lambda i: (i, 0)),
            pl.BlockSpec((1, K), lambda i: (0, 0)),
        ],
        out_specs=pl.BlockSpec((1, 1, BQ), lambda i: (i, 0, 0)),
        out_shape=jax.ShapeDtypeStruct((Q // BQ, 1, BQ), jnp.int32),
    )(xb, cb2, x2, c2row)
    return out.reshape(Q)
